# initial kernel scaffold (unmeasured)
import jax
import jax.numpy as jnp
from jax import lax
from jax.experimental import pallas as pl
from jax.experimental.pallas import tpu as pltpu

T = 2048
D = 4096
V_HALF = 8192
V = 16384

BT_MM = 512
BV_MM = 1024
BT_SM = 128


def _matmul_body(x_ref, w_ref, out_ref):
    out_ref[...] = jnp.dot(
        x_ref[...].astype(jnp.bfloat16),
        w_ref[...].astype(jnp.bfloat16),
        preferred_element_type=jnp.float32,
    )


def _local_matmul(x, w):
    return pl.pallas_call(
        _matmul_body,
        grid=(V_HALF // BV_MM, T // BT_MM),
        in_specs=[
            pl.BlockSpec((BT_MM, D), lambda j, i: (i, 0)),
            pl.BlockSpec((D, BV_MM), lambda j, i: (0, j)),
        ],
        out_specs=pl.BlockSpec((BT_MM, BV_MM), lambda j, i: (i, j)),
        out_shape=jax.ShapeDtypeStruct((T, V_HALF), jnp.float32),
    )(x, w)


def _exchange_softmax_body(
    logits_hbm, out_ref, recv_hbm,
    loc_vmem, rem_vmem, loc_sem, rem_sem, send_sem, recv_sem,
):
    i = pl.program_id(0)
    my_x = lax.axis_index("x")
    my_y = lax.axis_index("y")
    my_z = lax.axis_index("z")

    @pl.when(i == 0)
    def _():
        rdma = pltpu.make_async_remote_copy(
            src_ref=logits_hbm,
            dst_ref=recv_hbm,
            send_sem=send_sem,
            recv_sem=recv_sem,
            device_id=(my_x, my_y, 1 - my_z),
            device_id_type=pl.DeviceIdType.MESH,
        )
        rdma.start()
        rdma.wait()

    rows = pl.ds(i * BT_SM, BT_SM)
    cp_loc = pltpu.make_async_copy(logits_hbm.at[rows, :], loc_vmem, loc_sem)
    cp_rem = pltpu.make_async_copy(recv_hbm.at[rows, :], rem_vmem, rem_sem)
    cp_loc.start()
    cp_rem.start()
    cp_loc.wait()
    cp_rem.wait()

    loc = loc_vmem[...]
    rem = rem_vmem[...]
    m = jnp.maximum(
        jnp.max(loc, axis=1, keepdims=True),
        jnp.max(rem, axis=1, keepdims=True),
    )
    e_loc = jnp.exp(loc - m)
    e_rem = jnp.exp(rem - m)
    s = (
        jnp.sum(e_loc, axis=1, keepdims=True)
        + jnp.sum(e_rem, axis=1, keepdims=True)
    )
    p_loc = e_loc / s
    p_rem = e_rem / s

    @pl.when(my_z == 0)
    def _():
        out_ref[:, :V_HALF] = p_loc
        out_ref[:, V_HALF:] = p_rem

    @pl.when(my_z == 1)
    def _():
        out_ref[:, :V_HALF] = p_rem
        out_ref[:, V_HALF:] = p_loc


def _exchange_softmax(logits):
    out, _ = pl.pallas_call(
        _exchange_softmax_body,
        grid=(T // BT_SM,),
        in_specs=[pl.BlockSpec(memory_space=pltpu.MemorySpace.HBM)],
        out_specs=[
            pl.BlockSpec((BT_SM, V), lambda i: (i, 0)),
            pl.BlockSpec(memory_space=pltpu.MemorySpace.HBM),
        ],
        out_shape=[
            jax.ShapeDtypeStruct((T, V), jnp.float32),
            jax.ShapeDtypeStruct((T, V_HALF), jnp.float32),
        ],
        scratch_shapes=[
            pltpu.VMEM((BT_SM, V_HALF), jnp.float32),
            pltpu.VMEM((BT_SM, V_HALF), jnp.float32),
            pltpu.SemaphoreType.DMA,
            pltpu.SemaphoreType.DMA,
            pltpu.SemaphoreType.DMA,
            pltpu.SemaphoreType.DMA,
        ],
        compiler_params=pltpu.CompilerParams(
            dimension_semantics=("arbitrary",),
            collective_id=0,
        ),
    )(logits)
    return out


def kernel(x, W):
    logits = _local_matmul(x, W)
    return _exchange_softmax(logits)


# baseline (device time: 1158925 ns/iter reference)
import jax
import jax.numpy as jnp
from jax import lax
from jax.experimental import pallas as pl
from jax.experimental.pallas import tpu as pltpu

T = 2048
D = 4096
V_HALF = 8192
V = 16384

BT_MM = 512
BV_MM = 1024
BT_SM = 128


def _matmul_body(x_ref, w_ref, out_ref):
    out_ref[...] = jnp.dot(
        x_ref[...].astype(jnp.bfloat16),
        w_ref[...].astype(jnp.bfloat16),
        preferred_element_type=jnp.float32,
    )


def _local_matmul(x, w):
    return pl.pallas_call(
        _matmul_body,
        grid=(V_HALF // BV_MM, T // BT_MM),
        in_specs=[
            pl.BlockSpec((BT_MM, D), lambda j, i: (i, 0)),
            pl.BlockSpec((D, BV_MM), lambda j, i: (0, j)),
        ],
        out_specs=pl.BlockSpec((BT_MM, BV_MM), lambda j, i: (i, j)),
        out_shape=jax.ShapeDtypeStruct((T, V_HALF), jnp.float32),
        compiler_params=pltpu.CompilerParams(
            vmem_limit_bytes=64 * 1024 * 1024,
        ),
    )(x, w)


def _exchange_softmax_body(
    logits_hbm, out_ref, recv_hbm,
    loc_vmem, rem_vmem, loc_sem, rem_sem, send_sem, recv_sem,
):
    i = pl.program_id(0)
    my_x = lax.axis_index("x")
    my_y = lax.axis_index("y")
    my_z = lax.axis_index("z")

    @pl.when(i == 0)
    def _():
        rdma = pltpu.make_async_remote_copy(
            src_ref=logits_hbm,
            dst_ref=recv_hbm,
            send_sem=send_sem,
            recv_sem=recv_sem,
            device_id=(my_x, my_y, 1 - my_z),
            device_id_type=pl.DeviceIdType.MESH,
        )
        rdma.start()
        rdma.wait()

    rows = pl.ds(i * BT_SM, BT_SM)
    cp_loc = pltpu.make_async_copy(logits_hbm.at[rows, :], loc_vmem, loc_sem)
    cp_rem = pltpu.make_async_copy(recv_hbm.at[rows, :], rem_vmem, rem_sem)
    cp_loc.start()
    cp_rem.start()
    cp_loc.wait()
    cp_rem.wait()

    loc = loc_vmem[...]
    rem = rem_vmem[...]
    m = jnp.maximum(
        jnp.max(loc, axis=1, keepdims=True),
        jnp.max(rem, axis=1, keepdims=True),
    )
    e_loc = jnp.exp(loc - m)
    e_rem = jnp.exp(rem - m)
    s = (
        jnp.sum(e_loc, axis=1, keepdims=True)
        + jnp.sum(e_rem, axis=1, keepdims=True)
    )
    p_loc = e_loc / s
    p_rem = e_rem / s

    @pl.when(my_z == 0)
    def _():
        out_ref[:, :V_HALF] = p_loc
        out_ref[:, V_HALF:] = p_rem

    @pl.when(my_z == 1)
    def _():
        out_ref[:, :V_HALF] = p_rem
        out_ref[:, V_HALF:] = p_loc


def _exchange_softmax(logits):
    out, _ = pl.pallas_call(
        _exchange_softmax_body,
        grid=(T // BT_SM,),
        in_specs=[pl.BlockSpec(memory_space=pltpu.MemorySpace.HBM)],
        out_specs=[
            pl.BlockSpec((BT_SM, V), lambda i: (i, 0)),
            pl.BlockSpec(memory_space=pltpu.MemorySpace.HBM),
        ],
        out_shape=[
            jax.ShapeDtypeStruct((T, V), jnp.float32),
            jax.ShapeDtypeStruct((T, V_HALF), jnp.float32),
        ],
        scratch_shapes=[
            pltpu.VMEM((BT_SM, V_HALF), jnp.float32),
            pltpu.VMEM((BT_SM, V_HALF), jnp.float32),
            pltpu.SemaphoreType.DMA,
            pltpu.SemaphoreType.DMA,
            pltpu.SemaphoreType.DMA,
            pltpu.SemaphoreType.DMA,
        ],
        compiler_params=pltpu.CompilerParams(
            dimension_semantics=("arbitrary",),
            vmem_limit_bytes=64 * 1024 * 1024,
        ),
    )(logits)
    return out


def kernel(x, W):
    logits = _local_matmul(x, W)
    return _exchange_softmax(logits)


# device time: 453736 ns/iter; 2.5542x vs baseline; 2.5542x over previous
import jax
import jax.numpy as jnp
from jax import lax
from jax.experimental import pallas as pl
from jax.experimental.pallas import tpu as pltpu

T = 2048
D = 4096
V_HALF = 8192
V = 16384

BT_MM = 512
BV_MM = 512
NT = T // BT_MM
NV = V_HALF // BV_MM
BT_SM = 128


def _mm_send_body(
    x_ref, w_ref, loc_ref, recv_hbm, send_buf, send_sems, recv_sems
):
    j = pl.program_id(0)
    i = pl.program_id(1)
    my_x = lax.axis_index("x")
    my_y = lax.axis_index("y")
    my_z = lax.axis_index("z")
    slot = lax.rem(j, 2)

    @pl.when((j == 0) & (i == 0))
    def _():
        barrier_sem = pltpu.get_barrier_semaphore()
        pl.semaphore_signal(
            barrier_sem,
            inc=1,
            device_id=(my_x, my_y, 1 - my_z),
            device_id_type=pl.DeviceIdType.MESH,
        )
        pl.semaphore_wait(barrier_sem, 1)

    @pl.when((i == 0) & (j >= 2))
    def _():
        pltpu.make_async_remote_copy(
            src_ref=send_buf.at[slot],
            dst_ref=recv_hbm.at[:, pl.ds(j * BV_MM, BV_MM)],
            send_sem=send_sems.at[slot],
            recv_sem=recv_sems.at[j],
            device_id=(my_x, my_y, 1 - my_z),
            device_id_type=pl.DeviceIdType.MESH,
        ).wait_send()

    acc = jnp.dot(
        x_ref[...].astype(jnp.bfloat16),
        w_ref[...].astype(jnp.bfloat16),
        preferred_element_type=jnp.float32,
    ).astype(jnp.bfloat16)
    loc_ref[...] = acc
    send_buf[slot, pl.ds(i * BT_MM, BT_MM), :] = acc

    @pl.when(i == NT - 1)
    def _():
        pltpu.make_async_remote_copy(
            src_ref=send_buf.at[slot],
            dst_ref=recv_hbm.at[:, pl.ds(j * BV_MM, BV_MM)],
            send_sem=send_sems.at[slot],
            recv_sem=recv_sems.at[j],
            device_id=(my_x, my_y, 1 - my_z),
            device_id_type=pl.DeviceIdType.MESH,
        ).start()

    @pl.when((j == NV - 1) & (i == NT - 1))
    def _():
        for jj in (NV - 2, NV - 1):
            pltpu.make_async_remote_copy(
                src_ref=send_buf.at[jj % 2],
                dst_ref=recv_hbm.at[:, pl.ds(jj * BV_MM, BV_MM)],
                send_sem=send_sems.at[jj % 2],
                recv_sem=recv_sems.at[jj],
                device_id=(my_x, my_y, 1 - my_z),
                device_id_type=pl.DeviceIdType.MESH,
            ).wait_send()
        for jj in range(NV):
            pltpu.make_async_remote_copy(
                src_ref=send_buf.at[jj % 2],
                dst_ref=recv_hbm.at[:, pl.ds(jj * BV_MM, BV_MM)],
                send_sem=send_sems.at[jj % 2],
                recv_sem=recv_sems.at[jj],
                device_id=(my_x, my_y, 1 - my_z),
                device_id_type=pl.DeviceIdType.MESH,
            ).wait_recv()


def _mm_send(x, w):
    return pl.pallas_call(
        _mm_send_body,
        grid=(NV, NT),
        in_specs=[
            pl.BlockSpec((BT_MM, D), lambda j, i: (i, 0)),
            pl.BlockSpec((D, BV_MM), lambda j, i: (0, j)),
        ],
        out_specs=[
            pl.BlockSpec((BT_MM, BV_MM), lambda j, i: (i, j)),
            pl.BlockSpec(memory_space=pltpu.MemorySpace.HBM),
        ],
        out_shape=[
            jax.ShapeDtypeStruct((T, V_HALF), jnp.bfloat16),
            jax.ShapeDtypeStruct((T, V_HALF), jnp.bfloat16),
        ],
        scratch_shapes=[
            pltpu.VMEM((2, T, BV_MM), jnp.bfloat16),
            pltpu.SemaphoreType.DMA((2,)),
            pltpu.SemaphoreType.DMA((NV,)),
        ],
        compiler_params=pltpu.CompilerParams(
            dimension_semantics=("arbitrary", "arbitrary"),
            vmem_limit_bytes=64 * 1024 * 1024,
            collective_id=0,
        ),
    )(x, w)


def _softmax_body(loc_ref, rem_ref, out_ref):
    my_z = lax.axis_index("z")
    loc = loc_ref[...].astype(jnp.float32)
    rem = rem_ref[...].astype(jnp.float32)
    m = jnp.maximum(
        jnp.max(loc, axis=1, keepdims=True),
        jnp.max(rem, axis=1, keepdims=True),
    )
    e_loc = jnp.exp(loc - m)
    e_rem = jnp.exp(rem - m)
    s = (
        jnp.sum(e_loc, axis=1, keepdims=True)
        + jnp.sum(e_rem, axis=1, keepdims=True)
    )
    p_loc = e_loc / s
    p_rem = e_rem / s

    @pl.when(my_z == 0)
    def _():
        out_ref[:, :V_HALF] = p_loc
        out_ref[:, V_HALF:] = p_rem

    @pl.when(my_z == 1)
    def _():
        out_ref[:, :V_HALF] = p_rem
        out_ref[:, V_HALF:] = p_loc


def _softmax(loc, rem):
    return pl.pallas_call(
        _softmax_body,
        grid=(T // BT_SM,),
        in_specs=[
            pl.BlockSpec((BT_SM, V_HALF), lambda r: (r, 0)),
            pl.BlockSpec((BT_SM, V_HALF), lambda r: (r, 0)),
        ],
        out_specs=pl.BlockSpec((BT_SM, V), lambda r: (r, 0)),
        out_shape=jax.ShapeDtypeStruct((T, V), jnp.float32),
        compiler_params=pltpu.CompilerParams(
            dimension_semantics=("arbitrary",),
            vmem_limit_bytes=64 * 1024 * 1024,
        ),
    )(loc, rem)


def kernel(x, W):
    loc, rem = _mm_send(x, W)
    return _softmax(loc, rem)


# device time: 383748 ns/iter; 3.0200x vs baseline; 1.1824x over previous
import jax
import jax.numpy as jnp
from jax import lax
from jax.experimental import pallas as pl
from jax.experimental.pallas import tpu as pltpu

T = 2048
D = 4096
V_HALF = 8192
V = 16384

BT_MM = 512
BV_MM = 512
NT = T // BT_MM
NV = V_HALF // BV_MM
NK = NV // 2
BT_SM = 128


def _mm_send_body(
    x_ref, w_ref, loc_ref, recv_hbm,
    send_buf, zsend_sems, fsend_sems, zrecv_sems, xrecv_sems,
):
    j = pl.program_id(0)
    i = pl.program_id(1)
    my_x = lax.axis_index("x")
    my_y = lax.axis_index("y")
    my_z = lax.axis_index("z")
    zp = (my_x, my_y, 1 - my_z)
    xp = (1 - my_x, my_y, my_z)

    k = lax.div(j, 2)
    sslot = lax.rem(k, 2)
    is_mine = lax.rem(j, 2) == my_x

    def z_desc(block_j, kk, slot):
        return pltpu.make_async_remote_copy(
            src_ref=send_buf.at[slot],
            dst_ref=recv_hbm.at[:, pl.ds(block_j * BV_MM, BV_MM)],
            send_sem=zsend_sems.at[slot],
            recv_sem=zrecv_sems.at[kk],
            device_id=zp,
            device_id_type=pl.DeviceIdType.MESH,
        )

    def fwd_desc(block_j, kk, slot):
        return pltpu.make_async_remote_copy(
            src_ref=recv_hbm.at[:, pl.ds(block_j * BV_MM, BV_MM)],
            dst_ref=recv_hbm.at[:, pl.ds(block_j * BV_MM, BV_MM)],
            send_sem=fsend_sems.at[slot],
            recv_sem=xrecv_sems.at[kk],
            device_id=xp,
            device_id_type=pl.DeviceIdType.MESH,
        )

    @pl.when((j == 0) & (i == 0))
    def _():
        barrier_sem = pltpu.get_barrier_semaphore()
        for peer in (zp, xp):
            pl.semaphore_signal(
                barrier_sem,
                inc=1,
                device_id=peer,
                device_id_type=pl.DeviceIdType.MESH,
            )
        pl.semaphore_wait(barrier_sem, 2)

    @pl.when((i == 0) & is_mine & (k >= 2))
    def _():
        z_desc(j, k, sslot).wait_send()

    acc = jnp.dot(
        x_ref[...].astype(jnp.bfloat16),
        w_ref[...].astype(jnp.bfloat16),
        preferred_element_type=jnp.float32,
    ).astype(jnp.bfloat16)
    loc_ref[...] = acc

    @pl.when(is_mine)
    def _():
        send_buf[sslot, pl.ds(i * BT_MM, BT_MM), :] = acc

    @pl.when((i == NT - 1) & is_mine)
    def _():
        z_desc(j, k, sslot).start()

        @pl.when(k >= 1)
        def _():
            kf = k - 1
            jf = j - 2
            fslot = lax.rem(kf, 2)
            z_desc(jf, kf, fslot).wait_recv()

            @pl.when(kf >= 2)
            def _():
                fwd_desc(jf, kf, fslot).wait_send()

            fwd_desc(jf, kf, fslot).start()

    @pl.when((j == NV - 1) & (i == NT - 1))
    def _():
        jl = 2 * (NK - 1) + my_x
        z_desc(jl, NK - 1, (NK - 1) % 2).wait_recv()
        fwd_desc(jl, NK - 1, (NK - 1) % 2).wait_send()
        fwd_desc(jl, NK - 1, (NK - 1) % 2).start()

        z_desc(jl, NK - 1, 0).wait_send()
        z_desc(jl, NK - 1, 1).wait_send()
        fwd_desc(2 * (NK - 2) + my_x, NK - 2, (NK - 2) % 2).wait_send()
        fwd_desc(jl, NK - 1, (NK - 1) % 2).wait_send()

        for kk in range(NK):
            jx = 2 * kk + (1 - my_x)
            fwd_desc(jx, kk, 0).wait_recv()


def _mm_send(x, w):
    return pl.pallas_call(
        _mm_send_body,
        grid=(NV, NT),
        in_specs=[
            pl.BlockSpec((BT_MM, D), lambda j, i: (i, 0)),
            pl.BlockSpec((D, BV_MM), lambda j, i: (0, j)),
        ],
        out_specs=[
            pl.BlockSpec((BT_MM, BV_MM), lambda j, i: (i, j)),
            pl.BlockSpec(memory_space=pltpu.MemorySpace.HBM),
        ],
        out_shape=[
            jax.ShapeDtypeStruct((T, V_HALF), jnp.bfloat16),
            jax.ShapeDtypeStruct((T, V_HALF), jnp.bfloat16),
        ],
        scratch_shapes=[
            pltpu.VMEM((2, T, BV_MM), jnp.bfloat16),
            pltpu.SemaphoreType.DMA((2,)),
            pltpu.SemaphoreType.DMA((2,)),
            pltpu.SemaphoreType.DMA((NK,)),
            pltpu.SemaphoreType.DMA((NK,)),
        ],
        compiler_params=pltpu.CompilerParams(
            dimension_semantics=("arbitrary", "arbitrary"),
            vmem_limit_bytes=64 * 1024 * 1024,
            collective_id=0,
        ),
    )(x, w)


def _softmax_body(loc_ref, rem_ref, out_ref):
    my_z = lax.axis_index("z")
    loc = loc_ref[...].astype(jnp.float32)
    rem = rem_ref[...].astype(jnp.float32)
    m = jnp.maximum(
        jnp.max(loc, axis=1, keepdims=True),
        jnp.max(rem, axis=1, keepdims=True),
    )
    e_loc = jnp.exp(loc - m)
    e_rem = jnp.exp(rem - m)
    s = (
        jnp.sum(e_loc, axis=1, keepdims=True)
        + jnp.sum(e_rem, axis=1, keepdims=True)
    )
    p_loc = e_loc / s
    p_rem = e_rem / s

    @pl.when(my_z == 0)
    def _():
        out_ref[:, :V_HALF] = p_loc
        out_ref[:, V_HALF:] = p_rem

    @pl.when(my_z == 1)
    def _():
        out_ref[:, :V_HALF] = p_rem
        out_ref[:, V_HALF:] = p_loc


def _softmax(loc, rem):
    return pl.pallas_call(
        _softmax_body,
        grid=(T // BT_SM,),
        in_specs=[
            pl.BlockSpec((BT_SM, V_HALF), lambda r: (r, 0)),
            pl.BlockSpec((BT_SM, V_HALF), lambda r: (r, 0)),
        ],
        out_specs=pl.BlockSpec((BT_SM, V), lambda r: (r, 0)),
        out_shape=jax.ShapeDtypeStruct((T, V), jnp.float32),
        compiler_params=pltpu.CompilerParams(
            dimension_semantics=("arbitrary",),
            vmem_limit_bytes=64 * 1024 * 1024,
        ),
    )(loc, rem)


def kernel(x, W):
    loc, rem = _mm_send(x, W)
    return _softmax(loc, rem)
